# Initial kernel scaffold; baseline (speedup 1.0000x reference)
import jax
import jax.numpy as jnp
from jax.experimental import pallas as pl


def kernel(features, edge_index, edge_weight, lin_w, lin_b, bias):
    raise NotImplementedError("write your pallas kernel here")


# speedup vs baseline: 6.2227x; 6.2227x over previous
import functools

import jax
import jax.numpy as jnp
import numpy as np
from jax import lax
from jax.experimental import pallas as pl
from jax.experimental.pallas import tpu as pltpu
from jax.experimental.pallas import tpu_sc as plsc

N = 10000
E = 160000
IN_F = 300
OUT_F = 150
OUTP = 160
W2 = 2 * OUTP
DEG_COL = OUT_F

NC = 2
NS = 16
NW = NC * NS
NP = 10240
ROWS_PER_TILE = NP // NS
EDGES_PER_TILE = E // NW
CHUNK = 50
NCHUNK = EDGES_PER_TILE // CHUNK

DEG_CHUNK = 8
DEG_LANE = 11

BM = 1000


def _perm_cols():
    p = np.arange(OUTP)
    g, r = p // 32, p % 32
    return g * 32 + 2 * (r % 16) + (r // 16)


def _matmul_body(x_ref, w_ref, o_ref):
    o_ref[...] = jnp.dot(x_ref[...], w_ref[...],
                         preferred_element_type=jnp.float32)


def _matmul(features, wcat):
    return pl.pallas_call(
        _matmul_body,
        grid=(N // BM,),
        in_specs=[pl.BlockSpec((BM, IN_F), lambda i: (i, 0)),
                  pl.BlockSpec((IN_F, W2), lambda i: (0, 0))],
        out_specs=pl.BlockSpec((BM, W2), lambda i: (i, 0)),
        out_shape=jax.ShapeDtypeStruct((N, W2), jnp.float32),
    )(features, wcat)


def _sc_edges(zcat, src, dst, ew, zinit):
    mesh = plsc.VectorSubcoreMesh(core_axis_name="c", subcore_axis_name="s")

    @functools.partial(
        pl.kernel,
        out_type=jax.ShapeDtypeStruct((NC, NP, OUTP), jnp.bfloat16),
        mesh=mesh,
        compiler_params=pltpu.CompilerParams(use_tc_tiling_on_sc=False,
                                             needs_layout_passes=False),
        scratch_types=[
            pltpu.VMEM_SHARED((NP, OUTP), jnp.bfloat16),
            pltpu.VMEM((NCHUNK + 1, CHUNK), jnp.int32),
            pltpu.VMEM((NCHUNK, CHUNK), jnp.int32),
            pltpu.VMEM((EDGES_PER_TILE + 16,), jnp.float32),
            pltpu.VMEM((CHUNK, W2), jnp.float32),
            pltpu.VMEM((CHUNK, W2), jnp.float32),
            pltpu.VMEM((CHUNK, OUTP), jnp.bfloat16),
            pltpu.VMEM((CHUNK, OUTP), jnp.bfloat16),
            pltpu.SemaphoreType.DMA,
            pltpu.SemaphoreType.DMA,
            pltpu.SemaphoreType.DMA,
        ],
    )
    def k(zcat_hbm, src_hbm, dst_hbm, ew_hbm, zinit_hbm, out_hbm,
          acc, srcall, dstall, wall, rows0, rows1, mbuf0, mbuf1,
          gsem0, gsem1, ssem):
        c = lax.axis_index("c")
        s = lax.axis_index("s")
        wid = c * NS + s

        pltpu.sync_copy(zinit_hbm,
                        acc.at[pl.ds(s * ROWS_PER_TILE, ROWS_PER_TILE)])
        pltpu.sync_copy(src_hbm.at[wid], srcall.at[pl.ds(0, NCHUNK)])
        pltpu.sync_copy(src_hbm.at[wid, 0], srcall.at[NCHUNK])
        pltpu.sync_copy(dst_hbm.at[wid], dstall)
        pltpu.sync_copy(ew_hbm.at[wid], wall.at[pl.ds(0, EDGES_PER_TILE)])
        plsc.subcore_barrier()

        lanes = lax.iota(jnp.int32, 16)
        deg_onehot = jnp.where(lanes == DEG_LANE,
                               jnp.float32(1.0), jnp.float32(0.0))

        def start_gather(kk, rows, gsem):
            pltpu.async_copy(zcat_hbm.at[srcall.at[kk]], rows, gsem)

        def wait_gather(kk, rows, gsem):
            pltpu.make_async_copy(zcat_hbm.at[srcall.at[kk]], rows, gsem).wait()

        def compute(kk, rows, mbuf):
            def edge_body(e, carry2):
                wb = jnp.full((16,), wall[pl.ds(kk * CHUNK + e, 16)][0])
                m = []
                for cidx in range(OUTP // 16):
                    za = rows[e, pl.ds(cidx * 16, 16)]
                    zb = rows[e, pl.ds(OUTP + cidx * 16, 16)]
                    mc = wb * za + zb
                    if cidx == DEG_CHUNK:
                        mc = mc + deg_onehot
                    m.append(mc)
                for g in range(OUTP // 32):
                    mbuf[e, pl.ds(g * 32, 32)] = plsc.pack(
                        m[2 * g], m[2 * g + 1],
                        format=plsc.PackFormat.INTERLEAVED)
                return carry2

            lax.fori_loop(0, CHUNK, edge_body, 0)

        def start_scatter(kk, mbuf):
            pltpu.async_copy(mbuf, acc.at[dstall.at[kk]], ssem, add=True)

        def wait_scatter(kk, mbuf):
            pltpu.make_async_copy(mbuf, acc.at[dstall.at[kk]], ssem).wait()

        def process(kk, rows_cur, gsem_cur, mbuf_cur,
                    rows_nxt, gsem_nxt):
            wait_gather(kk, rows_cur, gsem_cur)
            start_gather(kk + 1, rows_nxt, gsem_nxt)

            @pl.when(kk > 1)
            def _():
                wait_scatter(kk - 2, mbuf_cur)

            compute(kk, rows_cur, mbuf_cur)
            start_scatter(kk, mbuf_cur)

        start_gather(0, rows0, gsem0)

        def pair_body(p, carry):
            process(2 * p, rows0, gsem0, mbuf0, rows1, gsem1)
            process(2 * p + 1, rows1, gsem1, mbuf1, rows0, gsem0)
            return carry

        lax.fori_loop(0, NCHUNK // 2, pair_body, 0)

        wait_gather(NCHUNK, rows0, gsem0)
        wait_scatter(NCHUNK - 2, mbuf0)
        wait_scatter(NCHUNK - 1, mbuf1)
        plsc.subcore_barrier()

        pltpu.sync_copy(acc.at[pl.ds(s * ROWS_PER_TILE, ROWS_PER_TILE)],
                        out_hbm.at[c, pl.ds(s * ROWS_PER_TILE, ROWS_PER_TILE)])

    return k(zcat, src, dst, ew, zinit)


def _finalize_body(p_ref, b_ref, o_ref):
    ssum = (p_ref[0].astype(jnp.float32)
            + p_ref[1].astype(jnp.float32))
    col = lax.broadcasted_iota(jnp.int32, (BM, OUTP), 1)
    deg = jnp.sum(jnp.where(col == DEG_COL, ssum, 0.0), axis=1,
                  keepdims=True)
    deg = jnp.maximum(deg, 1.0)
    o_ref[...] = jnp.maximum(ssum[:, :OUT_F] / deg + b_ref[...], 0.0)


def _finalize(part, bias2d):
    return pl.pallas_call(
        _finalize_body,
        grid=(N // BM,),
        in_specs=[pl.BlockSpec((NC, BM, OUTP), lambda i: (0, i, 0)),
                  pl.BlockSpec((1, OUT_F), lambda i: (0, 0))],
        out_specs=pl.BlockSpec((BM, OUT_F), lambda i: (i, 0)),
        out_shape=jax.ShapeDtypeStruct((N, OUT_F), jnp.float32),
    )(part, bias2d)


def kernel(features, edge_index, edge_weight, lin_w, lin_b, bias):
    wmat = lin_w[:, 0].reshape(IN_F, OUT_F)
    bmat = lin_b.reshape(IN_F, OUT_F)
    wcat = jnp.zeros((IN_F, W2), jnp.float32)
    wcat = wcat.at[:, :OUT_F].set(wmat)
    wcat = wcat.at[:, OUTP:OUTP + OUT_F].set(bmat)
    order = _perm_cols()
    order_full = np.concatenate([order, OUTP + order])
    wcat = jnp.take(wcat, jnp.asarray(order_full), axis=1)

    zcat = _matmul(features, wcat)

    src = edge_index[0].reshape(NW, NCHUNK, CHUNK)
    dst = edge_index[1].reshape(NW, NCHUNK, CHUNK)
    ew = edge_weight[:, 0].reshape(NW, EDGES_PER_TILE)
    zinit = jnp.zeros((ROWS_PER_TILE, OUTP), jnp.bfloat16)

    part = _sc_edges(zcat, src, dst, ew, zinit)
    return _finalize(part, bias.reshape(1, OUT_F))
